# Initial kernel scaffold; baseline (speedup 1.0000x reference)
import functools

import jax
import jax.numpy as jnp
from jax import lax
from jax.experimental import pallas as pl
from jax.experimental.pallas import tpu as pltpu
from jax.experimental.pallas import tpu_sc as plsc

OUT_VARS = 16
NUM_CORES = 2
NUM_SUBCORES = 16
NUM_WORKERS = NUM_CORES * NUM_SUBCORES
CHUNK = 3200


@functools.lru_cache(maxsize=None)
def _make_gather_clamp(n_rows: int, n_chunks: int):
  mesh = plsc.VectorSubcoreMesh(core_axis_name="c", subcore_axis_name="s")

  @functools.partial(
      pl.kernel,
      mesh=mesh,
      out_type=jax.ShapeDtypeStruct((n_rows, OUT_VARS), jnp.float32),
      scratch_types=[
          pltpu.VMEM((CHUNK,), jnp.int32),
          pltpu.VMEM((CHUNK, OUT_VARS), jnp.float32),
          pltpu.SemaphoreType.DMA,
      ],
  )
  def gather_clamp(idx_hbm, table_hbm, out_hbm, idx_v, rows_v, sem):
    wid = lax.axis_index("s") * NUM_CORES + lax.axis_index("c")
    rows_per_w = n_chunks * CHUNK
    base = wid * rows_per_w

    def chunk_body(ci, carry):
      off = base + ci * CHUNK
      pltpu.sync_copy(idx_hbm.at[pl.ds(off, CHUNK)], idx_v)
      pltpu.async_copy(table_hbm.at[idx_v], rows_v, sem).wait()

      def clamp_body(i, c):
        rows_v[i, :] = jnp.minimum(jnp.maximum(rows_v[i, :], 0.0), 1.0)
        return c

      lax.fori_loop(0, CHUNK, clamp_body, 0)
      pltpu.sync_copy(rows_v, out_hbm.at[pl.ds(off, CHUNK)])
      return carry

    lax.fori_loop(0, n_chunks, chunk_body, 0)

  return gather_clamp


def kernel(x, table):
  assert x.ndim == 2
  b, h = x.shape
  n_rows = b * h
  flat = x.reshape(n_rows).astype(jnp.int32)
  grain = NUM_WORKERS * CHUNK
  n_pad = (-n_rows) % grain
  if n_pad:
    flat = jnp.concatenate([flat, jnp.zeros((n_pad,), jnp.int32)])
  n_total = n_rows + n_pad
  out = _make_gather_clamp(n_total, n_total // grain)(flat, table)
  if n_pad:
    out = out[:n_rows]
  return out.reshape(b, h, OUT_VARS)


# speedup vs baseline: 2.1024x; 2.1024x over previous
import functools

import jax
import jax.numpy as jnp
from jax import lax
from jax.experimental import pallas as pl
from jax.experimental.pallas import tpu as pltpu
from jax.experimental.pallas import tpu_sc as plsc

OUT_VARS = 16
NUM_CORES = 2
NUM_SUBCORES = 16
NUM_WORKERS = NUM_CORES * NUM_SUBCORES
CHUNK = 3200


@functools.lru_cache(maxsize=None)
def _make_gather_clamp(n_rows: int, n_chunks: int):
  mesh = plsc.VectorSubcoreMesh(core_axis_name="c", subcore_axis_name="s")

  @functools.partial(
      pl.kernel,
      mesh=mesh,
      out_type=jax.ShapeDtypeStruct((n_rows, OUT_VARS), jnp.float32),
      scratch_types=[
          pltpu.VMEM((CHUNK,), jnp.int32),
          pltpu.VMEM((CHUNK, OUT_VARS), jnp.float32),
          pltpu.SemaphoreType.DMA,
      ],
      compiler_params=pltpu.CompilerParams(use_tc_tiling_on_sc=False),
  )
  def gather_clamp(idx_hbm, table_hbm, out_hbm, idx_v, rows_v, sem):
    wid = lax.axis_index("s") * NUM_CORES + lax.axis_index("c")
    rows_per_w = n_chunks * CHUNK
    base = wid * rows_per_w

    def chunk_body(ci, carry):
      off = base + ci * CHUNK
      pltpu.sync_copy(idx_hbm.at[pl.ds(off, CHUNK)], idx_v)
      pltpu.async_copy(table_hbm.at[idx_v], rows_v, sem).wait()

      def clamp_body(i, c):
        rows_v[i, :] = jnp.minimum(jnp.maximum(rows_v[i, :], 0.0), 1.0)
        return c

      lax.fori_loop(0, CHUNK, clamp_body, 0)
      pltpu.sync_copy(rows_v, out_hbm.at[pl.ds(off, CHUNK)])
      return carry

    lax.fori_loop(0, n_chunks, chunk_body, 0)

  return gather_clamp


def kernel(x, table):
  assert x.ndim == 2
  b, h = x.shape
  n_rows = b * h
  flat = x.reshape(n_rows).astype(jnp.int32)
  grain = NUM_WORKERS * CHUNK
  n_pad = (-n_rows) % grain
  if n_pad:
    flat = jnp.concatenate([flat, jnp.zeros((n_pad,), jnp.int32)])
  n_total = n_rows + n_pad
  out = _make_gather_clamp(n_total, n_total // grain)(flat, table)
  if n_pad:
    out = out[:n_rows]
  return out.reshape(b, h, OUT_VARS)


# speedup vs baseline: 2.4421x; 1.1615x over previous
import functools

import jax
import jax.numpy as jnp
from jax import lax
from jax.experimental import pallas as pl
from jax.experimental.pallas import tpu as pltpu
from jax.experimental.pallas import tpu_sc as plsc

OUT_VARS = 16
NUM_CORES = 2
NUM_SUBCORES = 16
NUM_WORKERS = NUM_CORES * NUM_SUBCORES
CHUNK = 3200


PACK = 128 // OUT_VARS


@functools.lru_cache(maxsize=None)
def _make_gather_clamp(n_rows: int, n_chunks: int):
  mesh = plsc.VectorSubcoreMesh(core_axis_name="c", subcore_axis_name="s")

  @functools.partial(
      pl.kernel,
      mesh=mesh,
      out_type=jax.ShapeDtypeStruct((n_rows // PACK, 128), jnp.float32),
      scratch_types=[
          pltpu.VMEM((CHUNK,), jnp.int32),
          pltpu.VMEM((CHUNK, OUT_VARS), jnp.float32),
          pltpu.VMEM((CHUNK // PACK, 128), jnp.float32),
          pltpu.SemaphoreType.DMA,
      ],
      compiler_params=pltpu.CompilerParams(use_tc_tiling_on_sc=False),
  )
  def gather_clamp(idx_hbm, table_hbm, out_hbm, idx_v, rows_v, pack_v, sem):
    wid = lax.axis_index("s") * NUM_CORES + lax.axis_index("c")
    rows_per_w = n_chunks * CHUNK
    base = wid * rows_per_w

    def chunk_body(ci, carry):
      off = base + ci * CHUNK
      pltpu.sync_copy(idx_hbm.at[pl.ds(off, CHUNK)], idx_v)
      pltpu.async_copy(table_hbm.at[idx_v], rows_v, sem).wait()

      def clamp_body(j, c):
        for k in range(PACK):
          row = rows_v[j * PACK + k, :]
          pack_v[j, k * OUT_VARS:(k + 1) * OUT_VARS] = (
              jnp.minimum(jnp.maximum(row, 0.0), 1.0))
        return c

      lax.fori_loop(0, CHUNK // PACK, clamp_body, 0)
      pltpu.sync_copy(pack_v, out_hbm.at[pl.ds(off // PACK, CHUNK // PACK)])
      return carry

    lax.fori_loop(0, n_chunks, chunk_body, 0)

  return gather_clamp


def kernel(x, table):
  assert x.ndim == 2
  b, h = x.shape
  n_rows = b * h
  flat = x.reshape(n_rows).astype(jnp.int32)
  grain = NUM_WORKERS * CHUNK
  n_pad = (-n_rows) % grain
  if n_pad:
    flat = jnp.concatenate([flat, jnp.zeros((n_pad,), jnp.int32)])
  n_total = n_rows + n_pad
  out = _make_gather_clamp(n_total, n_total // grain)(flat, table)
  out = out.reshape(n_total, OUT_VARS)
  if n_pad:
    out = out[:n_rows]
  return out.reshape(b, h, OUT_VARS)


# speedup vs baseline: 3.8632x; 1.5819x over previous
import functools

import jax
import jax.numpy as jnp
from jax import lax
from jax.experimental import pallas as pl
from jax.experimental.pallas import tpu as pltpu
from jax.experimental.pallas import tpu_sc as plsc

OUT_VARS = 16
NUM_CORES = 2
NUM_SUBCORES = 16
NUM_WORKERS = NUM_CORES * NUM_SUBCORES
H_BLOCK = 4
VT = OUT_VARS // 8


@functools.lru_cache(maxsize=None)
def _make_gather_clamp_t(hist: int, batch: int):
  cols = batch // NUM_WORKERS
  nbt = cols // 128
  nbt_total = batch // 128
  n_steps = hist // H_BLOCK
  mesh = plsc.VectorSubcoreMesh(core_axis_name="c", subcore_axis_name="s")

  @functools.partial(
      pl.kernel,
      mesh=mesh,
      out_type=jax.ShapeDtypeStruct((hist, VT * nbt_total * 8, 128),
                                    jnp.float32),
      compiler_params=pltpu.CompilerParams(
          needs_layout_passes=False, use_tc_tiling_on_sc=False),
      scratch_types=(
          [pltpu.VMEM((cols,), jnp.int32) for _ in range(H_BLOCK)] + [
              pltpu.VMEM((H_BLOCK * cols, OUT_VARS), jnp.float32),
              pltpu.VMEM((H_BLOCK * VT * nbt * 8, 128), jnp.float32),
              pltpu.SemaphoreType.DMA,
          ]),
  )
  def gather_clamp_t(xt_hbm, table_hbm, out_hbm, *scratch):
    idx_vs = scratch[:H_BLOCK]
    rows_v, pack_v, sem = scratch[H_BLOCK:]
    wid = lax.axis_index("s") * NUM_CORES + lax.axis_index("c")
    col0 = wid * cols
    bt0 = wid * nbt

    def step_body(si, carry):
      h0 = si * H_BLOCK
      icopies = [
          pltpu.async_copy(
              xt_hbm.at[h0 + hh, pl.ds(col0, cols)], idx_vs[hh], sem)
          for hh in range(H_BLOCK)]
      for c in icopies:
        c.wait()
      copies = [
          pltpu.async_copy(
              table_hbm.at[idx_vs[hh]],
              rows_v.at[pl.ds(hh * cols, cols)], sem)
          for hh in range(H_BLOCK)]
      for c in copies:
        c.wait()

      def pack_body(t, c):
        btl = t // 8
        bq = t % 8
        for hh in range(H_BLOCK):
          row_ids = hh * cols + t * 16 + lax.iota(jnp.int32, 16)
          for v in range(OUT_VARS):
            col_ids = jnp.full((16,), v, jnp.int32)
            vals = plsc.load_gather(rows_v, [row_ids, col_ids])
            vals = jnp.minimum(jnp.maximum(vals, 0.0), 1.0)
            prow = (hh * VT + v // 8) * (nbt * 8) + btl * 8 + v % 8
            pack_v[prow, pl.ds(bq * 16, 16)] = vals
        return c

      lax.fori_loop(0, cols // 16, pack_body, 0)
      ocopies = []
      for hh in range(H_BLOCK):
        for vt in range(VT):
          ocopies.append(pltpu.async_copy(
              pack_v.at[pl.ds((hh * VT + vt) * (nbt * 8), nbt * 8), :],
              out_hbm.at[h0 + hh,
                         pl.ds(vt * nbt_total * 8 + bt0 * 8, nbt * 8), :],
              sem))
      for c in ocopies:
        c.wait()
      return carry

    lax.fori_loop(0, n_steps, step_body, 0)

  return gather_clamp_t


def kernel(x, table):
  assert x.ndim == 2
  b, h = x.shape
  n_terms, d = table.shape
  assert d == OUT_VARS
  assert b % (128 * NUM_WORKERS) == 0 and h % H_BLOCK == 0
  xt = x.T.astype(jnp.int32)
  p = _make_gather_clamp_t(h, b)(xt, table)
  p5 = p.reshape(h, VT, b // 128, 8, 128)
  return p5.transpose(2, 4, 0, 1, 3).reshape(b, h, OUT_VARS)


# speedup vs baseline: 4.5872x; 1.1874x over previous
import functools

import jax
import jax.numpy as jnp
from jax import lax
from jax.experimental import pallas as pl
from jax.experimental.pallas import tpu as pltpu
from jax.experimental.pallas import tpu_sc as plsc

OUT_VARS = 16
NUM_CORES = 2
NUM_SUBCORES = 16
NUM_WORKERS = NUM_CORES * NUM_SUBCORES
H_BLOCK = 2
VT = OUT_VARS // 8


@functools.lru_cache(maxsize=None)
def _make_gather_clamp_t(hist: int, batch: int):
  cols = batch // NUM_WORKERS
  nbt = cols // 128
  nbt_total = batch // 128
  n_steps = hist // H_BLOCK
  assert n_steps % 2 == 0 and n_steps >= 6
  mesh = plsc.VectorSubcoreMesh(core_axis_name="c", subcore_axis_name="s")

  @functools.partial(
      pl.kernel,
      mesh=mesh,
      out_type=jax.ShapeDtypeStruct((hist, VT * nbt_total * 8, 128),
                                    jnp.float32),
      compiler_params=pltpu.CompilerParams(
          needs_layout_passes=False, use_tc_tiling_on_sc=False),
      scratch_types=(
          [pltpu.VMEM((cols,), jnp.int32) for _ in range(2 * H_BLOCK)] + [
              pltpu.VMEM((H_BLOCK * cols, OUT_VARS), jnp.float32),
              pltpu.VMEM((H_BLOCK * cols, OUT_VARS), jnp.float32),
              pltpu.VMEM((H_BLOCK * VT * nbt * 8, 128), jnp.float32),
              pltpu.VMEM((H_BLOCK * VT * nbt * 8, 128), jnp.float32),
              pltpu.SemaphoreType.DMA,
              pltpu.SemaphoreType.DMA,
              pltpu.SemaphoreType.DMA,
              pltpu.SemaphoreType.DMA,
          ]),
  )
  def gather_clamp_t(xt_hbm, table_hbm, out_hbm, *scratch):
    idx_vs = scratch[:2 * H_BLOCK]
    rows_v = scratch[2 * H_BLOCK:2 * H_BLOCK + 2]
    pack_v = scratch[2 * H_BLOCK + 2:2 * H_BLOCK + 4]
    idx_sem, gat_sem, out_sem0, out_sem1 = scratch[2 * H_BLOCK + 4:]
    out_sems = (out_sem0, out_sem1)
    wid = lax.axis_index("s") * NUM_CORES + lax.axis_index("c")
    col0 = wid * cols
    bt0 = wid * nbt

    def start_idx(s, slot):
      h0 = s * H_BLOCK
      for hh in range(H_BLOCK):
        pltpu.async_copy(
            xt_hbm.at[h0 + hh, pl.ds(col0, cols)],
            idx_vs[slot * H_BLOCK + hh], idx_sem)

    def wait_idx(slot):
      for hh in range(H_BLOCK):
        pltpu.make_async_copy(
            xt_hbm.at[0, pl.ds(col0, cols)],
            idx_vs[slot * H_BLOCK + hh], idx_sem).wait()

    def start_gather(slot):
      for hh in range(H_BLOCK):
        pltpu.async_copy(
            table_hbm.at[idx_vs[slot * H_BLOCK + hh]],
            rows_v[slot].at[pl.ds(hh * cols, cols)], gat_sem)

    def wait_gather(slot):
      pltpu.make_async_copy(
          table_hbm.at[pl.ds(0, H_BLOCK * cols)], rows_v[slot],
          gat_sem).wait()

    def pack(s, slot):
      rv, pv = rows_v[slot], pack_v[slot]

      def pack_body(t, c):
        btl = t // 8
        bq = t % 8
        for hh in range(H_BLOCK):
          row_ids = hh * cols + t * 16 + lax.iota(jnp.int32, 16)
          for v in range(OUT_VARS):
            col_ids = jnp.full((16,), v, jnp.int32)
            vals = plsc.load_gather(rv, [row_ids, col_ids])
            vals = jnp.minimum(jnp.maximum(vals, 0.0), 1.0)
            prow = (hh * VT + v // 8) * (nbt * 8) + btl * 8 + v % 8
            pv[prow, pl.ds(bq * 16, 16)] = vals
        return c

      lax.fori_loop(0, cols // 16, pack_body, 0)

    def start_out(s, slot):
      h0 = s * H_BLOCK
      for hh in range(H_BLOCK):
        for vt in range(VT):
          pltpu.async_copy(
              pack_v[slot].at[pl.ds((hh * VT + vt) * (nbt * 8), nbt * 8), :],
              out_hbm.at[h0 + hh,
                         pl.ds(vt * nbt_total * 8 + bt0 * 8, nbt * 8), :],
              out_sems[slot])

    def wait_out(slot):
      pltpu.make_async_copy(
          out_hbm.at[0, pl.ds(0, H_BLOCK * VT * nbt * 8), :],
          pack_v[slot], out_sems[slot]).wait()

    start_idx(0, 0)
    start_idx(1, 1)
    wait_idx(0)
    start_gather(0)
    wait_gather(0)
    wait_idx(1)
    start_gather(1)
    start_idx(2, 0)
    pack(0, 0)
    start_out(0, 0)
    wait_gather(1)
    wait_idx(0)
    start_gather(0)
    start_idx(3, 1)
    pack(1, 1)
    start_out(1, 1)

    def loop_body(g, carry):
      for b in range(2):
        s = 2 * g + b
        wait_gather(b)
        wait_idx(1 - b)
        start_gather(1 - b)
        start_idx(s + 2, b)
        wait_out(b)
        pack(s, b)
        start_out(s, b)
      return carry

    lax.fori_loop(1, n_steps // 2 - 1, loop_body, 0)

    s = n_steps - 2
    wait_gather(0)
    wait_idx(1)
    start_gather(1)
    wait_out(0)
    pack(s, 0)
    start_out(s, 0)
    wait_gather(1)
    wait_out(1)
    pack(s + 1, 1)
    start_out(s + 1, 1)
    wait_out(0)
    wait_out(1)

  return gather_clamp_t


def kernel(x, table):
  assert x.ndim == 2
  b, h = x.shape
  n_terms, d = table.shape
  assert d == OUT_VARS
  assert b % (128 * NUM_WORKERS) == 0 and h % H_BLOCK == 0
  xt = x.T.astype(jnp.int32)
  p = _make_gather_clamp_t(h, b)(xt, table)
  p5 = p.reshape(h, VT, b // 128, 8, 128)
  return p5.transpose(2, 4, 0, 1, 3).reshape(b, h, OUT_VARS)
